# Initial kernel scaffold; baseline (speedup 1.0000x reference)
import jax
import jax.numpy as jnp
from jax.experimental import pallas as pl


def kernel(x, pos_edge_index, edge_attr, target_mat, Wq1, bq1, Wk1, bk1, Wv1, bv1, We1, Ws1, bs1, gw1, gb1, gm1, Wq2, bq2, Wk2, bk2, Wv2, bv2, Ws2, bs2, gw2, gb2, gm2, D1, D2, D3):
    raise NotImplementedError("write your pallas kernel here")


# speedup vs baseline: 1045.6359x; 1045.6359x over previous
import functools
import math

import numpy as np
import jax
import jax.numpy as jnp
from jax.experimental import pallas as pl
from jax.experimental.pallas import tpu as pltpu

N1 = 160
N2 = 268
H = 4
C = 67
E_PAD = 25600
EB = 1024
NBLK = E_PAD // EB
SQ = 1.0 / math.sqrt(C)
R_TAYLOR = 20
M2 = N2 * (N2 - 1) // 2

_IU0, _IU1 = np.triu_indices(N2, 1)
_TRIU_FLAT = (_IU0 * N2 + _IU1).astype(np.int32)


def _dot(a, b, dims):
    return jax.lax.dot_general(a, b, (dims, ((), ())),
                               precision=jax.lax.Precision.HIGHEST,
                               preferred_element_type=jnp.float32)


def _stage1_kernel(x_ref, dst_ref, src_ref, ea_ref,
                   wq_ref, bq_ref, wk_ref, bk_ref, wv_ref, bv_ref,
                   we_ref, ws_ref, bs_ref, gw_ref, gb_ref, gm_ref,
                   t_ref, g_ref, gea_ref, sc_ref):
    x = x_ref[...]
    q = _dot(x, wq_ref[...], ((1,), (0,))) + bq_ref[...]
    k = _dot(x, wk_ref[...], ((1,), (0,))) + bk_ref[...]
    v = _dot(x, wv_ref[...], ((1,), (0,))) + bv_ref[...]
    skip = _dot(x, ws_ref[...], ((1,), (0,))) + bs_ref[...]
    we = we_ref[...]
    qw = q * we
    u = jnp.concatenate(
        [jnp.sum(qw[:, h * C:(h + 1) * C], axis=1, keepdims=True)
         for h in range(H)], axis=1)
    for h in range(H):
        sc_ref[h] = _dot(q[:, h * C:(h + 1) * C], k[:, h * C:(h + 1) * C],
                         ((1,), (1,)))

    g_ref[...] = jnp.zeros((H, N1, N1), jnp.float32)
    gea_ref[...] = jnp.zeros((H, N1, N1), jnp.float32)

    def body(j, carry):
        db = dst_ref[pl.ds(j, 1), :]
        sb = src_ref[pl.ds(j, 1), :]
        eb = ea_ref[pl.ds(j, 1), :]
        iota = jax.lax.broadcasted_iota(jnp.int32, (N1, EB), 0)
        ohd = (iota == db).astype(jnp.float32)
        ohs = (iota == sb).astype(jnp.float32)
        ugT = _dot(u, ohd, ((0,), (0,)))
        gT = jnp.exp(eb * ugT * SQ)
        for h in range(H):
            lhs = ohd * gT[h:h + 1, :]
            g_ref[h] = g_ref[h] + _dot(lhs, ohs, ((1,), (1,)))
            gea_ref[h] = gea_ref[h] + _dot(lhs * eb, ohs, ((1,), (1,)))
        return carry

    jax.lax.fori_loop(0, NBLK, body, 0)

    outs = []
    for h in range(H):
        Gh = g_ref[h]
        sch = sc_ref[h]
        rowmax = jnp.max(jnp.where(Gh > 0.0, sch, -1e30),
                         axis=1, keepdims=True)
        am = jnp.where(rowmax > -1e29,
                       rowmax * SQ + jnp.maximum(u[:, h:h + 1], 0.0) * SQ,
                       0.0)
        P = jnp.exp(sch * SQ - am)
        Wm = P * Gh
        denom = jnp.sum(Wm, axis=1, keepdims=True) + 1e-16
        numv = _dot(Wm, v[:, h * C:(h + 1) * C], ((1,), (0,)))
        numea = jnp.sum(P * gea_ref[h], axis=1, keepdims=True)
        outs.append((numv + numea * we[:, h * C:(h + 1) * C]) / denom)
    hfull = jnp.concatenate(outs, axis=1) + skip

    mean = jnp.mean(hfull, axis=0, keepdims=True)
    od = hfull - mean * gm_ref[...]
    var = jnp.mean(od * od, axis=0, keepdims=True)
    hn = gw_ref[...] * od / jnp.sqrt(var + 1e-5) + gb_ref[...]
    hn = jnp.maximum(hn, 0.0)

    xt = _dot(hn, hn, ((0,), (0,)))
    mn = jnp.min(xt, keepdims=True)
    mx = jnp.max(xt, keepdims=True)
    tm = (xt - mn) / (mx - mn + 1e-8)
    t_ref[...] = jnp.where(tm == 0.0, 1e-10, tm)


def _stage2_kernel(t_ref, sv_ref, tm_ref, d1_ref, d2_ref, d3_ref,
                   dn_ref, fake_ref, real_ref):
    T = t_ref[...]
    sv = sv_ref[...]
    aq, cq = sv[:, 0:1], sv[:, 1:2]
    ak, ck = sv[:, 2:3], sv[:, 3:4]
    av, cv = sv[:, 4:5], sv[:, 5:6]
    as_, cs = sv[:, 6:7], sv[:, 7:8]
    gw2, gb2, gm2 = sv[:, 8:9], sv[:, 9:10], sv[:, 10:11]
    A, B = aq * ak, aq * ck
    Cc, Dc = cq * ak, cq * ck

    ii = jax.lax.broadcasted_iota(jnp.int32, (N2, N2), 0)
    jj = jax.lax.broadcasted_iota(jnp.int32, (N2, N2), 1)
    eyeb = ii == jj
    mask = jnp.where(eyeb, 0.0, 1.0)

    Ec = jnp.exp(Cc * T) * mask
    Fc, Fr = [], []
    cur = Ec
    for r in range(R_TAYLOR + 2):
        Fc.append(jnp.sum(cur, axis=1, keepdims=True))
        Fr.append(jnp.sum(cur, axis=0, keepdims=True))
        if r < R_TAYLOR + 1:
            cur = cur * T
    AccZ = jnp.zeros((N2, N2), jnp.float32)
    AccN = jnp.zeros((N2, N2), jnp.float32)
    Pow = jnp.ones((N2, N2), jnp.float32)
    coef = jnp.ones((1, 1), jnp.float32)
    for r in range(R_TAYLOR + 1):
        FF = Fc[r] + Fr[r]
        HH = av * (Fc[r + 1] + Fr[r + 1]) + cv * FF
        AccZ = AccZ + coef * Pow * FF
        AccN = AccN + coef * Pow * HH
        if r < R_TAYLOR:
            Pow = Pow * T
            coef = coef * A / float(r + 1)
    pref = jnp.exp(B * T + Dc)
    selfE = jnp.exp(A * T * T + (B + Cc) * T + Dc)
    V = av * T + cv
    Z = pref * AccZ - 2.0 * selfE
    Nm = pref * AccN - 2.0 * V * selfE
    Dm = Nm / (Z + 1e-16) + as_ * T + cs

    dsum = jnp.sum(Dm * mask, keepdims=True) * (0.5 / M2)
    od = Dm - dsum * gm2
    var = jnp.sum(od * od * mask, keepdims=True) * (0.5 / M2)
    dn = gw2 * od / jnp.sqrt(var + 1e-5) + gb2
    dn = jnp.maximum(dn, 0.0)
    dmin = jnp.min(jnp.where(eyeb, 1e30, dn), keepdims=True)
    dmax = jnp.max(jnp.where(eyeb, -1e30, dn), keepdims=True)
    Dn = (dn - dmin) / (dmax - dmin + 1e-8)
    dn_ref[...] = Dn

    tm = tm_ref[...]
    eye_f = jnp.where(eyeb, 1.0, 0.0)
    tmT = _dot(tm, eye_f, ((0,), (0,)))
    pred = jnp.where(eyeb, 1.0, Dn)
    real = jnp.where(eyeb, 1.0, jnp.where(ii < jj, tm, tmT))
    zs = jnp.concatenate([pred, real], axis=0)
    a1 = jnp.maximum(_dot(zs, d1_ref[...], ((1,), (0,))), 0.0)
    a2 = jnp.maximum(_dot(a1, d2_ref[...], ((1,), (0,))), 0.0)
    lab = jnp.abs(jax.nn.sigmoid(_dot(a2, d3_ref[...], ((1,), (0,)))))
    fake_ref[...] = lab[0:N2, :]
    real_ref[...] = lab[N2:2 * N2, :]


def _impl(x, pos_edge_index, edge_attr, target_mat, Wq1, bq1, Wk1, bk1, Wv1,
          bv1, We1, Ws1, bs1, gw1, gb1, gm1, Wq2, bq2, Wk2, bk2, Wv2, bv2,
          Ws2, bs2, gw2, gb2, gm2, D1, D2, D3, interpret=False):
    E = pos_edge_index.shape[1]
    pad = E_PAD - E
    dstp = jnp.pad(pos_edge_index[1], (0, pad),
                   constant_values=N1).reshape(NBLK, EB)
    srcp = jnp.pad(pos_edge_index[0], (0, pad),
                   constant_values=N1).reshape(NBLK, EB)
    eap = jnp.pad(edge_attr[:, 0], (0, pad)).reshape(NBLK, EB)

    t_mat = pl.pallas_call(
        _stage1_kernel,
        out_shape=jax.ShapeDtypeStruct((N2, N2), jnp.float32),
        scratch_shapes=[
            pltpu.VMEM((H, N1, N1), jnp.float32),
            pltpu.VMEM((H, N1, N1), jnp.float32),
            pltpu.VMEM((H, N1, N1), jnp.float32),
        ],
        interpret=interpret,
    )(x, dstp, srcp, eap,
      Wq1, bq1[None, :], Wk1, bk1[None, :], Wv1, bv1[None, :],
      We1, Ws1, bs1[None, :], gw1[None, :], gb1[None, :], gm1[None, :])

    b0 = 2.0 / (N2 * (N2 - 1))
    sv = jnp.zeros((1, 16), jnp.float32)
    vals = [
        Wq2[0, 0], b0 * Wq2[1, 0] + Wq2[2, 0] + bq2[0],
        Wk2[0, 0], b0 * Wk2[1, 0] + Wk2[2, 0] + bk2[0],
        Wv2[0, 0], b0 * Wv2[1, 0] + Wv2[2, 0] + bv2[0],
        Ws2[0, 0], b0 * Ws2[1, 0] + Ws2[2, 0] + bs2[0],
        gw2[0], gb2[0], gm2[0],
    ]
    sv = sv.at[0, :len(vals)].set(jnp.stack(vals))

    dn, fake, reall = pl.pallas_call(
        _stage2_kernel,
        out_shape=[
            jax.ShapeDtypeStruct((N2, N2), jnp.float32),
            jax.ShapeDtypeStruct((N2, 1), jnp.float32),
            jax.ShapeDtypeStruct((N2, 1), jnp.float32),
        ],
        interpret=interpret,
    )(t_mat, sv, target_mat, D1, D2, D3)

    dual_pred_x = dn.reshape(-1)[_TRIU_FLAT][:, None]
    dual_target_x = target_mat.reshape(-1)[_TRIU_FLAT][:, None]
    return dual_pred_x, dual_target_x, fake, reall


def kernel(*args):
    return _impl(*args)
